# Initial kernel scaffold; baseline (speedup 1.0000x reference)
import jax
import jax.numpy as jnp
from jax.experimental import pallas as pl


def kernel(inputs, positional_embedding):
    raise NotImplementedError("write your pallas kernel here")


# speedup vs baseline: 1.5979x; 1.5979x over previous
import functools

import numpy as np
import jax
import jax.numpy as jnp
from jax import lax
from jax.experimental import pallas as pl
from jax.experimental.pallas import tpu as pltpu
from jax.experimental.pallas import tpu_sc as plsc

_NUM_HEADS = 16
_ATTN_HEIGHT = 24
_CLS_TOKEN_LEN = 1
_CLS_TOKEN_POS_LEN = 3

_S = 577
_N = _S * _S
_NW = 32
_PW = 10432
_NPAD = _NW * _PW
_CH = 2608
_K = _PW // _CH
_NRD = 2212


def _build_rpi_flat():
    height = _ATTN_HEIGHT
    width = (_S - _CLS_TOKEN_LEN) // height
    hh, ww = np.meshgrid(range(height), range(width))
    coords = np.stack([hh, ww], axis=-1)
    coords_flatten = np.reshape(coords, [-1, 2])
    relative_coords = coords_flatten[:, None, :] - coords_flatten[None, :, :]
    relative_coords_hh = relative_coords[:, :, 0] + height - 1
    relative_coords_ww = (relative_coords[:, :, 1] + width - 1) * (2 * height - 1)
    relative_coords = np.stack([relative_coords_hh, relative_coords_ww], axis=-1)
    rpi = np.sum(relative_coords, axis=-1).astype(np.int64)
    num_relative_distance = (2 * height - 1) * (2 * width - 1) + _CLS_TOKEN_POS_LEN
    top = np.full((1, rpi.shape[1]), num_relative_distance - 3, dtype=rpi.dtype)
    left = np.full((rpi.shape[0], 1), num_relative_distance - 2, dtype=rpi.dtype)
    corner = np.full((1, 1), num_relative_distance - 1, dtype=rpi.dtype)
    left_corner = np.concatenate([corner, left], axis=0)
    rpi = np.concatenate([top, rpi], axis=0)
    rpi = np.concatenate([left_corner, rpi], axis=1)
    rpi = rpi[:_S, :_S]
    flat = rpi.reshape(-1).astype(np.int32)
    return np.pad(flat, (0, _NPAD - _N))


_RPI_FLAT = _build_rpi_flat()


def _sc_gather(table, idx):
    mesh = plsc.VectorSubcoreMesh(core_axis_name="c", subcore_axis_name="s")
    info = plsc.get_sparse_core_info()
    nc = info.num_cores

    @functools.partial(
        pl.kernel,
        mesh=mesh,
        out_type=jax.ShapeDtypeStruct((_NUM_HEADS * _NPAD,), jnp.float32),
        scratch_types=[
            pltpu.VMEM((_NUM_HEADS * _NRD,), jnp.float32),
            pltpu.VMEM((_CH,), jnp.int32),
            pltpu.VMEM((_NUM_HEADS * _CH,), jnp.float32),
        ],
        compiler_params=pltpu.CompilerParams(
            use_tc_tiling_on_sc=False, needs_layout_passes=False
        ),
    )
    def sc_kernel(table_hbm, idx_hbm, out_hbm, tables_v, idx_v, outbuf_v):
        wid = lax.axis_index("s") * nc + lax.axis_index("c")
        base = wid * _PW
        pltpu.sync_copy(table_hbm, tables_v)

        def chunk_body(c, carry):
            off = base + c * _CH
            pltpu.sync_copy(idx_hbm.at[pl.ds(off, _CH)], idx_v)

            def gather_body(i, carry2):
                iv = idx_v[pl.ds(i * 16, 16)]
                for h in range(_NUM_HEADS):
                    outbuf_v[pl.ds(h * _CH + i * 16, 16)] = plsc.load_gather(
                        tables_v, [iv + (h * _NRD)]
                    )
                return carry2

            lax.fori_loop(0, _CH // 16, gather_body, 0)
            for h in range(_NUM_HEADS):
                pltpu.sync_copy(
                    outbuf_v.at[pl.ds(h * _CH, _CH)],
                    out_hbm.at[pl.ds(h * _NPAD + off, _CH)],
                )
            return carry

        lax.fori_loop(0, _K, chunk_body, 0)

    return sc_kernel(table.reshape(-1), idx)


_CHT = 16384
_NCH = -(-_N // _CHT)


def _add_body(x_ref, p_ref, o_ref):
    o_ref[...] = x_ref[...] + p_ref[...]


def _tc_add(in_flat, pos):
    batch = in_flat.shape[0]
    return pl.pallas_call(
        _add_body,
        grid=(_NCH, batch),
        in_specs=[
            pl.BlockSpec((1, _NUM_HEADS, _CHT), lambda c, b: (b, 0, c)),
            pl.BlockSpec((_NUM_HEADS, _CHT), lambda c, b: (0, c)),
        ],
        out_specs=pl.BlockSpec((1, _NUM_HEADS, _CHT), lambda c, b: (b, 0, c)),
        out_shape=jax.ShapeDtypeStruct(in_flat.shape, jnp.float32),
    )(in_flat, pos)


def kernel(inputs, positional_embedding):
    idx = jnp.asarray(_RPI_FLAT)
    pos = _sc_gather(positional_embedding, idx).reshape(_NUM_HEADS, _NPAD)
    in_flat = inputs.reshape(inputs.shape[0], _NUM_HEADS, _N)
    out = _tc_add(in_flat, pos)
    return out.reshape(inputs.shape)


# speedup vs baseline: 2.5113x; 1.5716x over previous
import functools

import numpy as np
import jax
import jax.numpy as jnp
from jax import lax
from jax.experimental import pallas as pl
from jax.experimental.pallas import tpu as pltpu
from jax.experimental.pallas import tpu_sc as plsc

_NUM_HEADS = 16
_ATTN_HEIGHT = 24
_CLS_TOKEN_LEN = 1
_CLS_TOKEN_POS_LEN = 3

_S = 577
_N = _S * _S
_NW = 32
_PW = 10432
_NPAD = _NW * _PW
_CH = 2608
_K = _PW // _CH
_NRD = 2212


def _build_rpi_flat():
    height = _ATTN_HEIGHT
    width = (_S - _CLS_TOKEN_LEN) // height
    hh, ww = np.meshgrid(range(height), range(width))
    coords = np.stack([hh, ww], axis=-1)
    coords_flatten = np.reshape(coords, [-1, 2])
    relative_coords = coords_flatten[:, None, :] - coords_flatten[None, :, :]
    relative_coords_hh = relative_coords[:, :, 0] + height - 1
    relative_coords_ww = (relative_coords[:, :, 1] + width - 1) * (2 * height - 1)
    relative_coords = np.stack([relative_coords_hh, relative_coords_ww], axis=-1)
    rpi = np.sum(relative_coords, axis=-1).astype(np.int64)
    num_relative_distance = (2 * height - 1) * (2 * width - 1) + _CLS_TOKEN_POS_LEN
    top = np.full((1, rpi.shape[1]), num_relative_distance - 3, dtype=rpi.dtype)
    left = np.full((rpi.shape[0], 1), num_relative_distance - 2, dtype=rpi.dtype)
    corner = np.full((1, 1), num_relative_distance - 1, dtype=rpi.dtype)
    left_corner = np.concatenate([corner, left], axis=0)
    rpi = np.concatenate([top, rpi], axis=0)
    rpi = np.concatenate([left_corner, rpi], axis=1)
    rpi = rpi[:_S, :_S]
    flat = rpi.reshape(-1).astype(np.int32)
    return np.pad(flat, (0, _NPAD - _N))


_RPI_FLAT = _build_rpi_flat()


def _sc_gather(table, idx):
    mesh = plsc.VectorSubcoreMesh(core_axis_name="c", subcore_axis_name="s")
    info = plsc.get_sparse_core_info()
    nc = info.num_cores

    @functools.partial(
        pl.kernel,
        mesh=mesh,
        out_type=jax.ShapeDtypeStruct((_NUM_HEADS * _NPAD,), jnp.float32),
        scratch_types=[
            pltpu.VMEM((_NUM_HEADS * _NRD,), jnp.float32),
            pltpu.VMEM((_CH,), jnp.int32),
            pltpu.VMEM((_NUM_HEADS * _CH,), jnp.float32),
        ],
        compiler_params=pltpu.CompilerParams(
            use_tc_tiling_on_sc=False, needs_layout_passes=False
        ),
    )
    def sc_kernel(table_hbm, idx_hbm, out_hbm, tables_v, idx_v, outbuf_v):
        wid = lax.axis_index("s") * nc + lax.axis_index("c")
        base = wid * _PW
        pltpu.sync_copy(table_hbm, tables_v)

        def chunk_body(c, carry):
            off = base + c * _CH
            pltpu.sync_copy(idx_hbm.at[pl.ds(off, _CH)], idx_v)

            def gather_body(i, carry2):
                iv = idx_v[pl.ds(i * 16, 16)]
                for h in range(_NUM_HEADS):
                    outbuf_v[pl.ds(h * _CH + i * 16, 16)] = plsc.load_gather(
                        tables_v, [iv + (h * _NRD)]
                    )
                return carry2

            lax.fori_loop(0, _CH // 16, gather_body, 0)
            for h in range(_NUM_HEADS):
                pltpu.sync_copy(
                    outbuf_v.at[pl.ds(h * _CH, _CH)],
                    out_hbm.at[pl.ds(h * _NPAD + off, _CH)],
                )
            return carry

        lax.fori_loop(0, _K, chunk_body, 0)

    return sc_kernel(table.reshape(-1), idx)


def _add_body(x_ref, p_ref, o_ref):
    o_ref[...] = x_ref[...] + p_ref[...]


def _tc_add(inputs, pos3):
    batch = inputs.shape[0]
    return pl.pallas_call(
        _add_body,
        grid=(_NUM_HEADS, batch),
        in_specs=[
            pl.BlockSpec((1, 1, _S, _S), lambda h, b: (b, h, 0, 0)),
            pl.BlockSpec((1, _S, _S), lambda h, b: (h, 0, 0)),
        ],
        out_specs=pl.BlockSpec((1, 1, _S, _S), lambda h, b: (b, h, 0, 0)),
        out_shape=jax.ShapeDtypeStruct(inputs.shape, jnp.float32),
    )(inputs, pos3)


def kernel(inputs, positional_embedding):
    idx = jnp.asarray(_RPI_FLAT)
    pos = _sc_gather(positional_embedding, idx)
    pos3 = pos.reshape(_NUM_HEADS, _NPAD)[:, :_N].reshape(_NUM_HEADS, _S, _S)
    return _tc_add(inputs, pos3)


# speedup vs baseline: 2.5808x; 1.0277x over previous
import functools

import numpy as np
import jax
import jax.numpy as jnp
from jax import lax
from jax.experimental import pallas as pl
from jax.experimental.pallas import tpu as pltpu
from jax.experimental.pallas import tpu_sc as plsc

_NUM_HEADS = 16
_ATTN_HEIGHT = 24
_CLS_TOKEN_LEN = 1
_CLS_TOKEN_POS_LEN = 3

_S = 577
_N = _S * _S
_NW = 32
_PW = 10496
_NPAD = _NW * _PW
_CH = 1312
_K = _PW // _CH
_NRD = 2212


def _build_rpi_flat():
    height = _ATTN_HEIGHT
    width = (_S - _CLS_TOKEN_LEN) // height
    hh, ww = np.meshgrid(range(height), range(width))
    coords = np.stack([hh, ww], axis=-1)
    coords_flatten = np.reshape(coords, [-1, 2])
    relative_coords = coords_flatten[:, None, :] - coords_flatten[None, :, :]
    relative_coords_hh = relative_coords[:, :, 0] + height - 1
    relative_coords_ww = (relative_coords[:, :, 1] + width - 1) * (2 * height - 1)
    relative_coords = np.stack([relative_coords_hh, relative_coords_ww], axis=-1)
    rpi = np.sum(relative_coords, axis=-1).astype(np.int64)
    num_relative_distance = (2 * height - 1) * (2 * width - 1) + _CLS_TOKEN_POS_LEN
    top = np.full((1, rpi.shape[1]), num_relative_distance - 3, dtype=rpi.dtype)
    left = np.full((rpi.shape[0], 1), num_relative_distance - 2, dtype=rpi.dtype)
    corner = np.full((1, 1), num_relative_distance - 1, dtype=rpi.dtype)
    left_corner = np.concatenate([corner, left], axis=0)
    rpi = np.concatenate([top, rpi], axis=0)
    rpi = np.concatenate([left_corner, rpi], axis=1)
    rpi = rpi[:_S, :_S]
    flat = rpi.reshape(-1).astype(np.int32)
    return np.pad(flat, (0, _NPAD - _N))


_RPI_FLAT = _build_rpi_flat()


def _sc_gather(table, idx):
    mesh = plsc.VectorSubcoreMesh(core_axis_name="c", subcore_axis_name="s")
    info = plsc.get_sparse_core_info()
    nc = info.num_cores

    @functools.partial(
        pl.kernel,
        mesh=mesh,
        out_type=jax.ShapeDtypeStruct((_NUM_HEADS * _NPAD,), jnp.float32),
        scratch_types=[
            pltpu.VMEM((_NUM_HEADS * _NRD,), jnp.float32),
            pltpu.VMEM((_CH,), jnp.int32),
            pltpu.VMEM((_CH,), jnp.int32),
            pltpu.VMEM((_NUM_HEADS * _CH,), jnp.float32),
            pltpu.VMEM((_NUM_HEADS * _CH,), jnp.float32),
            pltpu.SemaphoreType.DMA,
            pltpu.SemaphoreType.DMA,
            pltpu.SemaphoreType.DMA,
            pltpu.SemaphoreType.DMA,
        ],
        compiler_params=pltpu.CompilerParams(
            use_tc_tiling_on_sc=False, needs_layout_passes=False
        ),
    )
    def sc_kernel(table_hbm, idx_hbm, out_hbm, tables_v, idx0_v, idx1_v,
                  out0_v, out1_v, semi0, semi1, semo0, semo1):
        wid = lax.axis_index("s") * nc + lax.axis_index("c")
        base = wid * _PW
        pltpu.sync_copy(table_hbm, tables_v)

        idx_bufs = (idx0_v, idx1_v)
        out_bufs = (out0_v, out1_v)
        idx_sems = (semi0, semi1)
        out_sems = (semo0, semo1)

        def start_idx(c):
            return pltpu.async_copy(
                idx_hbm.at[pl.ds(base + c * _CH, _CH)],
                idx_bufs[c % 2],
                idx_sems[c % 2],
            )

        idx_cp = {0: start_idx(0)}
        out_cp = {}
        for c in range(_K):
            p = c % 2
            if c + 1 < _K:
                idx_cp[c + 1] = start_idx(c + 1)
            idx_cp[c].wait()
            if c >= 2:
                for cp in out_cp[c - 2]:
                    cp.wait()
            iv_ref = idx_bufs[p]
            ob = out_bufs[p]
            for h in range(_NUM_HEADS):
                hoff = h * _NRD
                hbase = h * _CH

                @plsc.parallel_loop(0, _CH // 16, step=1, unroll=4)
                def gbody(i, iv_ref=iv_ref, ob=ob, hoff=hoff, hbase=hbase):
                    iv = iv_ref[pl.ds(i * 16, 16)]
                    ob[pl.ds(hbase + i * 16, 16)] = plsc.load_gather(
                        tables_v, [iv + hoff]
                    )

            off = base + c * _CH
            cps = []
            for h in range(_NUM_HEADS):
                cps.append(
                    pltpu.async_copy(
                        ob.at[pl.ds(h * _CH, _CH)],
                        out_hbm.at[pl.ds(h * _NPAD + off, _CH)],
                        out_sems[p],
                    )
                )
            out_cp[c] = cps
        for c in (_K - 2, _K - 1):
            for cp in out_cp[c]:
                cp.wait()

    return sc_kernel(table.reshape(-1), idx)


def _add_body(x_ref, p_ref, o_ref):
    o_ref[...] = x_ref[...] + p_ref[...]


def _tc_add(inputs, pos3):
    batch = inputs.shape[0]
    return pl.pallas_call(
        _add_body,
        grid=(_NUM_HEADS, batch),
        in_specs=[
            pl.BlockSpec((1, 1, _S, _S), lambda h, b: (b, h, 0, 0)),
            pl.BlockSpec((1, _S, _S), lambda h, b: (h, 0, 0)),
        ],
        out_specs=pl.BlockSpec((1, 1, _S, _S), lambda h, b: (b, h, 0, 0)),
        out_shape=jax.ShapeDtypeStruct(inputs.shape, jnp.float32),
    )(inputs, pos3)


def kernel(inputs, positional_embedding):
    idx = jnp.asarray(_RPI_FLAT)
    pos = _sc_gather(positional_embedding, idx)
    pos3 = pos.reshape(_NUM_HEADS, _NPAD)[:, :_N].reshape(_NUM_HEADS, _S, _S)
    return _tc_add(inputs, pos3)


# speedup vs baseline: 3.3138x; 1.2840x over previous
import functools

import numpy as np
import jax
import jax.numpy as jnp
from jax import lax
from jax.experimental import pallas as pl
from jax.experimental.pallas import tpu as pltpu
from jax.experimental.pallas import tpu_sc as plsc

_NUM_HEADS = 16
_ATTN_HEIGHT = 24
_CLS_TOKEN_LEN = 1
_CLS_TOKEN_POS_LEN = 3

_S = 577
_PR = 592
_PC = 640
_TR = 74
_TRU = 8 * _PC
_NRD = 2212


def _build_rpi_perm():
    height = _ATTN_HEIGHT
    width = (_S - _CLS_TOKEN_LEN) // height
    hh, ww = np.meshgrid(range(height), range(width))
    coords = np.stack([hh, ww], axis=-1)
    coords_flatten = np.reshape(coords, [-1, 2])
    relative_coords = coords_flatten[:, None, :] - coords_flatten[None, :, :]
    relative_coords_hh = relative_coords[:, :, 0] + height - 1
    relative_coords_ww = (relative_coords[:, :, 1] + width - 1) * (2 * height - 1)
    relative_coords = np.stack([relative_coords_hh, relative_coords_ww], axis=-1)
    rpi = np.sum(relative_coords, axis=-1).astype(np.int64)
    num_relative_distance = (2 * height - 1) * (2 * width - 1) + _CLS_TOKEN_POS_LEN
    top = np.full((1, rpi.shape[1]), num_relative_distance - 3, dtype=rpi.dtype)
    left = np.full((rpi.shape[0], 1), num_relative_distance - 2, dtype=rpi.dtype)
    corner = np.full((1, 1), num_relative_distance - 1, dtype=rpi.dtype)
    left_corner = np.concatenate([corner, left], axis=0)
    rpi = np.concatenate([top, rpi], axis=0)
    rpi = np.concatenate([left_corner, rpi], axis=1)
    rpi = rpi[:_S, :_S].astype(np.int32)
    pad = np.zeros((_PR, _PC), dtype=np.int32)
    pad[:_S, :_S] = rpi
    return np.ascontiguousarray(pad.reshape(-1))


_RPI_PERM = _build_rpi_perm()


def _sc_gather(table, idx):
    mesh = plsc.VectorSubcoreMesh(core_axis_name="c", subcore_axis_name="s")
    info = plsc.get_sparse_core_info()
    nc = info.num_cores

    @functools.partial(
        pl.kernel,
        mesh=mesh,
        out_type=jax.ShapeDtypeStruct((_NUM_HEADS, _PR, _PC), jnp.float32),
        scratch_types=[
            pltpu.VMEM((_NUM_HEADS * _NRD,), jnp.float32),
            pltpu.VMEM((_TRU,), jnp.int32),
            pltpu.VMEM((_TRU,), jnp.int32),
            pltpu.VMEM((8, _PC), jnp.float32),
            pltpu.VMEM((8, _PC), jnp.float32),
            pltpu.SemaphoreType.DMA,
            pltpu.SemaphoreType.DMA,
            pltpu.SemaphoreType.DMA,
            pltpu.SemaphoreType.DMA,
        ],
        compiler_params=pltpu.CompilerParams(
            use_tc_tiling_on_sc=True, needs_layout_passes=False
        ),
    )
    def sc_kernel(table_hbm, idx_hbm, out_hbm, tables_v, idx0_v, idx1_v,
                  ob0_v, ob1_v, semi0, semi1, semo0, semo1):
        wid = lax.axis_index("s") * nc + lax.axis_index("c")
        pltpu.sync_copy(table_hbm, tables_v)

        def tile_row(c):
            return jnp.where(
                wid < 9,
                3 * wid + c,
                jnp.where(c < 2, 2 * wid + 9 + c, _TR - 1),
            )

        trs = [tile_row(c) for c in range(3)]
        idx_bufs = (idx0_v, idx1_v)
        out_bufs = (ob0_v, ob1_v)
        idx_sems = (semi0, semi1)
        out_sems = (semo0, semo1)

        def start_idx(c):
            return pltpu.async_copy(
                idx_hbm.at[pl.ds(trs[c] * _TRU, _TRU)],
                idx_bufs[c % 2],
                idx_sems[c % 2],
            )

        idx_cp = {0: start_idx(0)}
        out_cp = {}
        u = 0
        for c in range(3):
            if c + 1 < 3:
                idx_cp[c + 1] = start_idx(c + 1)
            idx_cp[c].wait()
            iv_ref = idx_bufs[c % 2]
            for h in range(_NUM_HEADS):
                p = u % 2
                if u >= 2:
                    out_cp[u - 2].wait()
                ob = out_bufs[p]
                hoff = h * _NRD

                @plsc.parallel_loop(0, _TRU // 16, step=1, unroll=4)
                def gbody(i, iv_ref=iv_ref, ob=ob, hoff=hoff):
                    r = i // (_PC // 16)
                    s = i % (_PC // 16)
                    iv = iv_ref[pl.ds(i * 16, 16)]
                    ob[r, pl.ds(s * 16, 16)] = plsc.load_gather(
                        tables_v, [iv + hoff]
                    )

                out_cp[u] = pltpu.async_copy(
                    ob,
                    out_hbm.at[h, pl.ds(trs[c] * 8, 8), :],
                    out_sems[p],
                )
                u += 1
        out_cp[u - 2].wait()
        out_cp[u - 1].wait()

    return sc_kernel(table.reshape(-1), idx)


def _add_body(x_ref, p_ref, o_ref):
    o_ref[0, 0] = x_ref[0, 0] + p_ref[0, :_S, :_S]


def _tc_add(inputs, pos):
    batch = inputs.shape[0]
    return pl.pallas_call(
        _add_body,
        grid=(_NUM_HEADS, batch),
        in_specs=[
            pl.BlockSpec((1, 1, _S, _S), lambda h, b: (b, h, 0, 0)),
            pl.BlockSpec((1, _PR, _PC), lambda h, b: (h, 0, 0)),
        ],
        out_specs=pl.BlockSpec((1, 1, _S, _S), lambda h, b: (b, h, 0, 0)),
        out_shape=jax.ShapeDtypeStruct(inputs.shape, jnp.float32),
    )(inputs, pos)


def kernel(inputs, positional_embedding):
    idx = jnp.asarray(_RPI_PERM)
    pos = _sc_gather(positional_embedding, idx)
    return _tc_add(inputs, pos)
